# baseline (speedup 1.0000x reference)
import jax
import jax.numpy as jnp
from jax import lax
from jax.experimental import pallas as pl
from jax.experimental.pallas import tpu as pltpu
from jax.experimental.pallas import tpu_sc as plsc

D = 128
CHUNK = 128
NBUF = 2
NUM_CORES = 2
NUM_SUBCORES = 16
NTILES = NUM_CORES * NUM_SUBCORES
LANES = 16


def _sage_aggregate(x, src2d, dst2d, npad, with_cnt):
  rows_per_tile = npad // NUM_SUBCORES
  zchunks = rows_per_tile // CHUNK
  total_chunks = src2d.shape[0] // CHUNK
  cpt = total_chunks // NTILES

  out_types = [jax.ShapeDtypeStruct((NUM_CORES, npad, D), jnp.float32)]
  if with_cnt:
    out_types.append(jax.ShapeDtypeStruct((NUM_CORES, npad), jnp.float32))

  scratch = (
      [pltpu.VMEM((CHUNK, D), jnp.float32) for _ in range(NBUF)]
      + [pltpu.VMEM((CHUNK,), jnp.int32) for _ in range(NBUF)]
      + [pltpu.VMEM((CHUNK,), jnp.int32) for _ in range(NBUF)]
      + [
          pltpu.VMEM((CHUNK,), jnp.float32),
          pltpu.VMEM((CHUNK,), jnp.float32),
          pltpu.VMEM_SHARED((npad, D), jnp.float32),
          pltpu.VMEM_SHARED((npad,), jnp.float32),
      ]
      + [pltpu.SemaphoreType.DMA for _ in range(NBUF)]
      + [pltpu.SemaphoreType.DMA for _ in range(NBUF)]
      + [pltpu.SemaphoreType.DMA, pltpu.SemaphoreType.DMA]
  )
  mesh = plsc.VectorSubcoreMesh(core_axis_name="c", subcore_axis_name="s")

  def body(x_hbm, src_hbm, dst_hbm, *refs):
    it = iter(refs)
    agg_out = next(it)
    cnt_out = next(it) if with_cnt else None
    rows = [next(it) for _ in range(NBUF)]
    dbuf = [next(it) for _ in range(NBUF)]
    sbuf = [next(it) for _ in range(NBUF)]
    ones_v = next(it)
    zrow_v = next(it)
    agg_sh = next(it)
    cnt_sh = next(it)
    gsem = [next(it) for _ in range(NBUF)]
    dsem = [next(it) for _ in range(NBUF)]
    ssem = next(it)
    csem = next(it)

    c = lax.axis_index("c")
    s = lax.axis_index("s")
    wid = s * NUM_CORES + c
    row0 = s * rows_per_tile

    ebase = wid * cpt * CHUNK

    def fill_rows(i, _):
      r = i // (D // LANES)
      col = (i % (D // LANES)) * LANES
      rows[0][r, pl.ds(col, LANES)] = jnp.zeros((LANES,), jnp.float32)
      return 0
    lax.fori_loop(0, CHUNK * (D // LANES), fill_rows, 0)

    def fill_small(i, _):
      ones_v[pl.ds(i * LANES, LANES)] = jnp.ones((LANES,), jnp.float32)
      zrow_v[pl.ds(i * LANES, LANES)] = jnp.zeros((LANES,), jnp.float32)
      return 0
    lax.fori_loop(0, CHUNK // LANES, fill_small, 0)

    def zero_blk(k, _):
      pltpu.sync_copy(rows[0], agg_sh.at[pl.ds(row0 + k * CHUNK, CHUNK)])
      pltpu.sync_copy(zrow_v, cnt_sh.at[pl.ds(row0 + k * CHUNK, CHUNK)])
      return 0
    lax.fori_loop(0, zchunks, zero_blk, 0)
    plsc.subcore_barrier()

    def edge_step(j, _):
      off = ebase + j * CHUNK
      pltpu.sync_copy(src_hbm.at[pl.ds(off, CHUNK)], sbuf[0])
      pltpu.sync_copy(dst_hbm.at[pl.ds(off, CHUNK)], dbuf[0])
      pltpu.async_copy(x_hbm.at[sbuf[0]], rows[0], gsem[0]).wait()
      pltpu.sync_copy(rows[0], agg_sh.at[dbuf[0]], add=True)
      if with_cnt:
        pltpu.sync_copy(ones_v, cnt_sh.at[dbuf[0]], add=True)
      return 0
    lax.fori_loop(0, cpt, edge_step, 0)
    plsc.subcore_barrier()

    def writeback(k, _):
      r = row0 + k * CHUNK
      pltpu.sync_copy(agg_sh.at[pl.ds(r, CHUNK)], rows[0])
      pltpu.sync_copy(rows[0], agg_out.at[c, pl.ds(r, CHUNK)])
      if with_cnt:
        pltpu.sync_copy(cnt_sh.at[pl.ds(r, CHUNK)], zrow_v)
        pltpu.sync_copy(zrow_v, cnt_out.at[c, pl.ds(r, CHUNK)])
      return 0
    lax.fori_loop(0, zchunks, writeback, 0)

  res = pl.kernel(
      body,
      out_type=tuple(out_types),
      mesh=mesh,
      scratch_types=scratch,
  )(x, src2d, dst2d)
  if not with_cnt and isinstance(res, (tuple, list)):
    res = res[0]
  return res


def _sage_linear(agg_parts, cnt_col, x, Wl, b, Wr, relu):
  n = x.shape[0]
  bn = 2000
  grid = n // bn

  def body(agg_ref, cnt_ref, x_ref, wl_ref, b_ref, wr_ref, o_ref):
    mean = (agg_ref[0] + agg_ref[1]) / jnp.maximum(cnt_ref[...], 1.0)
    o = (jnp.dot(mean, wl_ref[...], preferred_element_type=jnp.float32)
         + b_ref[...]
         + jnp.dot(x_ref[...], wr_ref[...], preferred_element_type=jnp.float32))
    if relu:
      o = jnp.maximum(o, 0.0)
    o_ref[...] = o

  return pl.pallas_call(
      body,
      grid=(grid,),
      in_specs=[
          pl.BlockSpec((NUM_CORES, bn, D), lambda i: (0, i, 0)),
          pl.BlockSpec((bn, 1), lambda i: (i, 0)),
          pl.BlockSpec((bn, D), lambda i: (i, 0)),
          pl.BlockSpec((D, D), lambda i: (0, 0)),
          pl.BlockSpec((1, D), lambda i: (0, 0)),
          pl.BlockSpec((D, D), lambda i: (0, 0)),
      ],
      out_specs=pl.BlockSpec((bn, D), lambda i: (i, 0)),
      out_shape=jax.ShapeDtypeStruct((n, D), jnp.float32),
  )(agg_parts, cnt_col, x, Wl, b.reshape(1, D), Wr)


def kernel(x, edge_index, W1l, b1, W1r, W2l, b2, W2r):
  n = x.shape[0]
  e = edge_index.shape[1]
  rows_align = NUM_SUBCORES * CHUNK
  npad = ((n + 1 + rows_align - 1) // rows_align) * rows_align
  e_align = NTILES * CHUNK * NBUF
  e_pad = ((e + e_align - 1) // e_align) * e_align

  src = edge_index[0].astype(jnp.int32)
  dst = edge_index[1].astype(jnp.int32)
  src2d = jnp.concatenate([src, jnp.zeros((e_pad - e,), jnp.int32)])
  dst2d = jnp.concatenate([dst, jnp.full((e_pad - e,), n, jnp.int32)])

  agg1, cnt = _sage_aggregate(x, src2d, dst2d, npad, True)
  cnt_col = (cnt[0] + cnt[1]).reshape(npad, 1)
  h = _sage_linear(agg1, cnt_col, x, W1l, b1, W1r, relu=True)
  agg2 = _sage_aggregate(h, src2d, dst2d, npad, False)
  return _sage_linear(agg2, cnt_col, h, W2l, b2, W2r, relu=False)


# speedup vs baseline: 1.2440x; 1.2440x over previous
import jax
import jax.numpy as jnp
from jax import lax
from jax.experimental import pallas as pl
from jax.experimental.pallas import tpu as pltpu
from jax.experimental.pallas import tpu_sc as plsc

D = 128
CHUNK = 128
NBUF = 2
NUM_CORES = 2
NUM_SUBCORES = 16
NTILES = NUM_CORES * NUM_SUBCORES
LANES = 16


def _sage_aggregate(x, src2d, dst2d, npad, with_cnt):
  rows_per_tile = npad // NUM_SUBCORES
  zchunks = rows_per_tile // CHUNK
  total_chunks = src2d.shape[0] // CHUNK
  cpt = total_chunks // NTILES

  out_types = [jax.ShapeDtypeStruct((NUM_CORES, npad, D), jnp.float32)]
  if with_cnt:
    out_types.append(jax.ShapeDtypeStruct((NUM_CORES, npad), jnp.float32))

  scratch = (
      [pltpu.VMEM((CHUNK, D), jnp.float32) for _ in range(NBUF)]
      + [pltpu.VMEM((CHUNK,), jnp.int32) for _ in range(NBUF)]
      + [pltpu.VMEM((CHUNK,), jnp.int32) for _ in range(NBUF)]
      + [
          pltpu.VMEM((CHUNK,), jnp.float32),
          pltpu.VMEM((CHUNK,), jnp.float32),
          pltpu.VMEM_SHARED((npad, D), jnp.float32),
          pltpu.VMEM_SHARED((npad,), jnp.float32),
      ]
      + [pltpu.SemaphoreType.DMA for _ in range(NBUF)]
      + [pltpu.SemaphoreType.DMA for _ in range(NBUF)]
      + [pltpu.SemaphoreType.DMA, pltpu.SemaphoreType.DMA]
  )
  mesh = plsc.VectorSubcoreMesh(core_axis_name="c", subcore_axis_name="s")

  def body(x_hbm, src_hbm, dst_hbm, *refs):
    it = iter(refs)
    agg_out = next(it)
    cnt_out = next(it) if with_cnt else None
    rows = [next(it) for _ in range(NBUF)]
    dbuf = [next(it) for _ in range(NBUF)]
    sbuf = [next(it) for _ in range(NBUF)]
    ones_v = next(it)
    zrow_v = next(it)
    agg_sh = next(it)
    cnt_sh = next(it)
    gsem = [next(it) for _ in range(NBUF)]
    dsem = [next(it) for _ in range(NBUF)]
    ssem = next(it)
    csem = next(it)

    c = lax.axis_index("c")
    s = lax.axis_index("s")
    wid = s * NUM_CORES + c
    row0 = s * rows_per_tile

    ebase = wid * cpt * CHUNK

    def fill_rows(i, _):
      r = i // (D // LANES)
      col = (i % (D // LANES)) * LANES
      rows[0][r, pl.ds(col, LANES)] = jnp.zeros((LANES,), jnp.float32)
      return 0
    lax.fori_loop(0, CHUNK * (D // LANES), fill_rows, 0)

    def fill_small(i, _):
      ones_v[pl.ds(i * LANES, LANES)] = jnp.ones((LANES,), jnp.float32)
      zrow_v[pl.ds(i * LANES, LANES)] = jnp.zeros((LANES,), jnp.float32)
      return 0
    lax.fori_loop(0, CHUNK // LANES, fill_small, 0)

    def zero_blk(k, _):
      pltpu.sync_copy(rows[0], agg_sh.at[pl.ds(row0 + k * CHUNK, CHUNK)])
      pltpu.sync_copy(zrow_v, cnt_sh.at[pl.ds(row0 + k * CHUNK, CHUNK)])
      return 0
    lax.fori_loop(0, zchunks, zero_blk, 0)
    plsc.subcore_barrier()

    def prep(j, b):
      pltpu.async_copy(dst_hbm.at[pl.ds(ebase + j * CHUNK, CHUNK)],
                       dbuf[b], dsem[b])
      pltpu.sync_copy(src_hbm.at[pl.ds(ebase + j * CHUNK, CHUNK)], sbuf[b])
      pltpu.async_copy(x_hbm.at[sbuf[b]], rows[b], gsem[b])

    for b in range(NBUF):
      prep(b, b)

    def consume(j, b, prefetch):
      pltpu.make_async_copy(dst_hbm.at[pl.ds(ebase + j * CHUNK, CHUNK)],
                            dbuf[b], dsem[b]).wait()
      pltpu.make_async_copy(x_hbm.at[sbuf[b]], rows[b], gsem[b]).wait()
      sc = pltpu.async_copy(rows[b], agg_sh.at[dbuf[b]], ssem, add=True)
      if with_cnt:
        cc = pltpu.async_copy(ones_v, cnt_sh.at[dbuf[b]], csem, add=True)
      sc.wait()
      if with_cnt:
        cc.wait()
      if prefetch:
        prep(j + NBUF, b)

    def outer(jo, _):
      for b in range(NBUF):
        consume(jo * NBUF + b, b, True)
      return 0
    lax.fori_loop(0, cpt // NBUF - 1, outer, 0)
    for b in range(NBUF):
      consume(cpt - NBUF + b, b, False)
    plsc.subcore_barrier()

    def writeback(k, _):
      r = row0 + k * CHUNK
      pltpu.sync_copy(agg_sh.at[pl.ds(r, CHUNK)], rows[0])
      pltpu.sync_copy(rows[0], agg_out.at[c, pl.ds(r, CHUNK)])
      if with_cnt:
        pltpu.sync_copy(cnt_sh.at[pl.ds(r, CHUNK)], zrow_v)
        pltpu.sync_copy(zrow_v, cnt_out.at[c, pl.ds(r, CHUNK)])
      return 0
    lax.fori_loop(0, zchunks, writeback, 0)

  res = pl.kernel(
      body,
      out_type=tuple(out_types),
      mesh=mesh,
      scratch_types=scratch,
  )(x, src2d, dst2d)
  if not with_cnt and isinstance(res, (tuple, list)):
    res = res[0]
  return res


def _sage_linear(agg_parts, cnt_col, x, Wl, b, Wr, relu):
  n = x.shape[0]
  bn = 2000
  grid = n // bn

  def body(agg_ref, cnt_ref, x_ref, wl_ref, b_ref, wr_ref, o_ref):
    mean = (agg_ref[0] + agg_ref[1]) / jnp.maximum(cnt_ref[...], 1.0)
    o = (jnp.dot(mean, wl_ref[...], preferred_element_type=jnp.float32)
         + b_ref[...]
         + jnp.dot(x_ref[...], wr_ref[...], preferred_element_type=jnp.float32))
    if relu:
      o = jnp.maximum(o, 0.0)
    o_ref[...] = o

  return pl.pallas_call(
      body,
      grid=(grid,),
      in_specs=[
          pl.BlockSpec((NUM_CORES, bn, D), lambda i: (0, i, 0)),
          pl.BlockSpec((bn, 1), lambda i: (i, 0)),
          pl.BlockSpec((bn, D), lambda i: (i, 0)),
          pl.BlockSpec((D, D), lambda i: (0, 0)),
          pl.BlockSpec((1, D), lambda i: (0, 0)),
          pl.BlockSpec((D, D), lambda i: (0, 0)),
      ],
      out_specs=pl.BlockSpec((bn, D), lambda i: (i, 0)),
      out_shape=jax.ShapeDtypeStruct((n, D), jnp.float32),
  )(agg_parts, cnt_col, x, Wl, b.reshape(1, D), Wr)


def kernel(x, edge_index, W1l, b1, W1r, W2l, b2, W2r):
  n = x.shape[0]
  e = edge_index.shape[1]
  rows_align = NUM_SUBCORES * CHUNK
  npad = ((n + 1 + rows_align - 1) // rows_align) * rows_align
  e_align = NTILES * CHUNK * NBUF
  e_pad = ((e + e_align - 1) // e_align) * e_align

  src = edge_index[0].astype(jnp.int32)
  dst = edge_index[1].astype(jnp.int32)
  spare = jnp.arange(e_pad - e, dtype=jnp.int32) % (npad - n) + n
  src2d = jnp.concatenate([src, jnp.zeros((e_pad - e,), jnp.int32)])
  dst2d = jnp.concatenate([dst, spare])

  agg1, cnt = _sage_aggregate(x, src2d, dst2d, npad, True)
  cnt_col = (cnt[0] + cnt[1]).reshape(npad, 1)
  h = _sage_linear(agg1, cnt_col, x, W1l, b1, W1r, relu=True)
  agg2 = _sage_aggregate(h, src2d, dst2d, npad, False)
  return _sage_linear(agg2, cnt_col, h, W2l, b2, W2r, relu=False)


# speedup vs baseline: 4.0691x; 3.2710x over previous
import jax
import jax.numpy as jnp
from jax import lax
from jax.experimental import pallas as pl
from jax.experimental.pallas import tpu as pltpu
from jax.experimental.pallas import tpu_sc as plsc

D = 128
CHUNK = 128
NBUF = 2
NUM_CORES = 2
NUM_SUBCORES = 16
NTILES = NUM_CORES * NUM_SUBCORES
LANES = 16


def _sage_aggregate(x, src2d, dst2d, npad, with_cnt):
  rows_per_tile = npad // NUM_SUBCORES
  zchunks = rows_per_tile // CHUNK
  total_chunks = src2d.shape[0] // CHUNK
  cpt = total_chunks // NTILES

  out_types = [jax.ShapeDtypeStruct((NUM_CORES, npad, D), jnp.float32)]
  if with_cnt:
    out_types.append(jax.ShapeDtypeStruct((NUM_CORES, npad), jnp.float32))

  scratch = (
      [pltpu.VMEM((CHUNK, D), jnp.float32) for _ in range(NBUF)]
      + [pltpu.VMEM((CHUNK,), jnp.int32) for _ in range(NBUF)]
      + [pltpu.VMEM((CHUNK,), jnp.int32) for _ in range(NBUF)]
      + [
          pltpu.VMEM((CHUNK,), jnp.float32),
          pltpu.VMEM((CHUNK,), jnp.float32),
          pltpu.VMEM_SHARED((npad, D), jnp.float32),
          pltpu.VMEM_SHARED((npad,), jnp.float32),
      ]
      + [pltpu.SemaphoreType.DMA for _ in range(NBUF)]
      + [pltpu.SemaphoreType.DMA for _ in range(NBUF)]
      + [pltpu.SemaphoreType.DMA, pltpu.SemaphoreType.DMA]
  )
  mesh = plsc.VectorSubcoreMesh(core_axis_name="c", subcore_axis_name="s")

  def body(x_hbm, src_hbm, dst_hbm, *refs):
    it = iter(refs)
    agg_out = next(it)
    cnt_out = next(it) if with_cnt else None
    rows = [next(it) for _ in range(NBUF)]
    dbuf = [next(it) for _ in range(NBUF)]
    sbuf = [next(it) for _ in range(NBUF)]
    ones_v = next(it)
    zrow_v = next(it)
    agg_sh = next(it)
    cnt_sh = next(it)
    gsem = [next(it) for _ in range(NBUF)]
    dsem = [next(it) for _ in range(NBUF)]
    ssem = next(it)
    csem = next(it)

    c = lax.axis_index("c")
    s = lax.axis_index("s")
    wid = s * NUM_CORES + c
    row0 = s * rows_per_tile

    ebase = wid * cpt * CHUNK

    def fill_rows(i, _):
      r = i // (D // LANES)
      col = (i % (D // LANES)) * LANES
      rows[0][r, pl.ds(col, LANES)] = jnp.zeros((LANES,), jnp.float32)
      return 0
    lax.fori_loop(0, CHUNK * (D // LANES), fill_rows, 0)

    def fill_small(i, _):
      ones_v[pl.ds(i * LANES, LANES)] = jnp.ones((LANES,), jnp.float32)
      zrow_v[pl.ds(i * LANES, LANES)] = jnp.zeros((LANES,), jnp.float32)
      return 0
    lax.fori_loop(0, CHUNK // LANES, fill_small, 0)

    with jax.named_scope("zero_acc"):
      def zero_blk(k, _):
        pltpu.sync_copy(rows[0], agg_sh.at[pl.ds(row0 + k * CHUNK, CHUNK)])
        pltpu.sync_copy(zrow_v, cnt_sh.at[pl.ds(row0 + k * CHUNK, CHUNK)])
        return 0
      lax.fori_loop(0, zchunks, zero_blk, 0)
      plsc.subcore_barrier()

    def prep(j, b):
      pltpu.async_copy(dst_hbm.at[pl.ds(ebase + j * CHUNK, CHUNK)],
                       dbuf[b], dsem[b])
      pltpu.sync_copy(src_hbm.at[pl.ds(ebase + j * CHUNK, CHUNK)], sbuf[b])
      pltpu.async_copy(x_hbm.at[sbuf[b]], rows[b], gsem[b])

    for b in range(NBUF):
      prep(b, b)

    def consume(j, b, prefetch):
      pltpu.make_async_copy(dst_hbm.at[pl.ds(ebase + j * CHUNK, CHUNK)],
                            dbuf[b], dsem[b]).wait()
      pltpu.make_async_copy(x_hbm.at[sbuf[b]], rows[b], gsem[b]).wait()
      sc = pltpu.async_copy(rows[b], agg_sh.at[dbuf[b]], ssem, add=True)
      if with_cnt:
        cc = pltpu.async_copy(ones_v, cnt_sh.at[dbuf[b]], csem, add=True)
      sc.wait()
      if with_cnt:
        cc.wait()
      if prefetch:
        prep(j + NBUF, b)

    with jax.named_scope("edge_loop"):
      def outer(jo, _):
        for b in range(NBUF):
          consume(jo * NBUF + b, b, True)
        return 0
      lax.fori_loop(0, cpt // NBUF - 1, outer, 0)
      for b in range(NBUF):
        consume(cpt - NBUF + b, b, False)
      plsc.subcore_barrier()

    with jax.named_scope("writeback"):
      def writeback(k, _):
        r = row0 + k * CHUNK
        pltpu.sync_copy(agg_sh.at[pl.ds(r, CHUNK)], rows[0])
        pltpu.sync_copy(rows[0], agg_out.at[c, pl.ds(r, CHUNK)])
        if with_cnt:
          pltpu.sync_copy(cnt_sh.at[pl.ds(r, CHUNK)], zrow_v)
          pltpu.sync_copy(zrow_v, cnt_out.at[c, pl.ds(r, CHUNK)])
        return 0
      lax.fori_loop(0, zchunks, writeback, 0)

  res = pl.kernel(
      body,
      out_type=tuple(out_types),
      mesh=mesh,
      scratch_types=scratch,
  )(x, src2d, dst2d)
  if not with_cnt and isinstance(res, (tuple, list)):
    res = res[0]
  return res


def _sage_linear(agg_parts, cnt_col, x, Wl, b, Wr, relu):
  n = x.shape[0]
  bn = 2000
  grid = n // bn

  def body(agg_ref, cnt_ref, x_ref, wl_ref, b_ref, wr_ref, o_ref):
    mean = (agg_ref[0] + agg_ref[1]) / jnp.maximum(cnt_ref[...], 1.0)
    o = (jnp.dot(mean, wl_ref[...], preferred_element_type=jnp.float32)
         + b_ref[...]
         + jnp.dot(x_ref[...], wr_ref[...], preferred_element_type=jnp.float32))
    if relu:
      o = jnp.maximum(o, 0.0)
    o_ref[...] = o

  return pl.pallas_call(
      body,
      grid=(grid,),
      in_specs=[
          pl.BlockSpec((NUM_CORES, bn, D), lambda i: (0, i, 0)),
          pl.BlockSpec((bn, 1), lambda i: (i, 0)),
          pl.BlockSpec((bn, D), lambda i: (i, 0)),
          pl.BlockSpec((D, D), lambda i: (0, 0)),
          pl.BlockSpec((1, D), lambda i: (0, 0)),
          pl.BlockSpec((D, D), lambda i: (0, 0)),
      ],
      out_specs=pl.BlockSpec((bn, D), lambda i: (i, 0)),
      out_shape=jax.ShapeDtypeStruct((n, D), jnp.float32),
  )(agg_parts, cnt_col, x, Wl, b.reshape(1, D), Wr)


def kernel(x, edge_index, W1l, b1, W1r, W2l, b2, W2r):
  n = x.shape[0]
  e = edge_index.shape[1]
  rows_align = NUM_SUBCORES * CHUNK
  npad = ((n + 1 + rows_align - 1) // rows_align) * rows_align
  e_align = NTILES * CHUNK * NBUF
  e_pad = ((e + e_align - 1) // e_align) * e_align

  src = edge_index[0].astype(jnp.int32)
  dst = edge_index[1].astype(jnp.int32)
  pad_i = jnp.arange(e_pad - e, dtype=jnp.int32)
  src2d = jnp.concatenate([src, pad_i % n])
  dst2d = jnp.concatenate([dst, pad_i % (npad - n) + n])

  agg1, cnt = _sage_aggregate(x, src2d, dst2d, npad, True)
  cnt_col = (cnt[0] + cnt[1]).reshape(npad, 1)
  h = _sage_linear(agg1, cnt_col, x, W1l, b1, W1r, relu=True)
  agg2 = _sage_aggregate(h, src2d, dst2d, npad, False)
  return _sage_linear(agg2, cnt_col, h, W2l, b2, W2r, relu=False)


# speedup vs baseline: 4.5326x; 1.1139x over previous
import jax
import jax.numpy as jnp
from jax import lax
from jax.experimental import pallas as pl
from jax.experimental.pallas import tpu as pltpu
from jax.experimental.pallas import tpu_sc as plsc

D = 128
CHUNK = 128
NBUF = 2
NUM_CORES = 2
NUM_SUBCORES = 16
NTILES = NUM_CORES * NUM_SUBCORES
LANES = 16


def _sage_aggregate(x, src2d, dst2d, npad, with_cnt):
  rows_per_tile = npad // NUM_SUBCORES
  zchunks = rows_per_tile // CHUNK
  epc = src2d.shape[0] // NTILES
  cpt = epc // CHUNK
  tail = epc % CHUNK
  cpt_pipe = cpt - (cpt % NBUF)

  out_types = [jax.ShapeDtypeStruct((NUM_CORES, npad, D), jnp.float32)]
  if with_cnt:
    out_types.append(jax.ShapeDtypeStruct((NUM_CORES, npad), jnp.float32))

  scratch = (
      [pltpu.VMEM((CHUNK, D), jnp.float32) for _ in range(NBUF)]
      + [pltpu.VMEM((CHUNK,), jnp.int32) for _ in range(NBUF)]
      + [
          pltpu.VMEM((epc,), jnp.int32),
          pltpu.VMEM((max(tail, 1),), jnp.int32),
          pltpu.VMEM((CHUNK,), jnp.float32),
          pltpu.VMEM((CHUNK,), jnp.float32),
          pltpu.VMEM_SHARED((npad, D), jnp.float32),
          pltpu.VMEM_SHARED((npad,), jnp.float32),
      ]
      + [pltpu.SemaphoreType.DMA for _ in range(NBUF)]
      + [pltpu.SemaphoreType.DMA for _ in range(NBUF)]
      + [pltpu.SemaphoreType.DMA, pltpu.SemaphoreType.DMA]
  )
  mesh = plsc.VectorSubcoreMesh(core_axis_name="c", subcore_axis_name="s")

  def body(x_hbm, src_hbm, dst_hbm, *refs):
    it = iter(refs)
    agg_out = next(it)
    cnt_out = next(it) if with_cnt else None
    rows = [next(it) for _ in range(NBUF)]
    dbuf = [next(it) for _ in range(NBUF)]
    sidx = next(it)
    dtail = next(it)
    ones_v = next(it)
    zrow_v = next(it)
    agg_sh = next(it)
    cnt_sh = next(it)
    gsem = [next(it) for _ in range(NBUF)]
    dsem = [next(it) for _ in range(NBUF)]
    ssem = next(it)
    csem = next(it)

    c = lax.axis_index("c")
    s = lax.axis_index("s")
    wid = s * NUM_CORES + c
    row0 = s * rows_per_tile

    ebase = wid * epc
    ssd = pltpu.async_copy(src_hbm.at[pl.ds(ebase, epc)], sidx, gsem[0])

    def fill_rows(i, _):
      r = i // (D // LANES)
      col = (i % (D // LANES)) * LANES
      rows[0][r, pl.ds(col, LANES)] = jnp.zeros((LANES,), jnp.float32)
      return 0
    lax.fori_loop(0, CHUNK * (D // LANES), fill_rows, 0)

    def fill_small(i, _):
      ones_v[pl.ds(i * LANES, LANES)] = jnp.ones((LANES,), jnp.float32)
      zrow_v[pl.ds(i * LANES, LANES)] = jnp.zeros((LANES,), jnp.float32)
      return 0
    lax.fori_loop(0, CHUNK // LANES, fill_small, 0)

    with jax.named_scope("zero_acc"):
      def zero_blk(k, _):
        pltpu.sync_copy(rows[0], agg_sh.at[pl.ds(row0 + k * CHUNK, CHUNK)])
        pltpu.sync_copy(zrow_v, cnt_sh.at[pl.ds(row0 + k * CHUNK, CHUNK)])
        return 0
      lax.fori_loop(0, zchunks, zero_blk, 0)
      ssd.wait()
      plsc.subcore_barrier()

    def prep(j, b):
      pltpu.async_copy(dst_hbm.at[pl.ds(ebase + j * CHUNK, CHUNK)],
                       dbuf[b], dsem[b])
      pltpu.async_copy(x_hbm.at[sidx.at[pl.ds(j * CHUNK, CHUNK)]],
                       rows[b], gsem[b])

    def consume(j, b, prefetch):
      pltpu.make_async_copy(dst_hbm.at[pl.ds(ebase + j * CHUNK, CHUNK)],
                            dbuf[b], dsem[b]).wait()
      pltpu.make_async_copy(x_hbm.at[sidx.at[pl.ds(j * CHUNK, CHUNK)]],
                            rows[b], gsem[b]).wait()
      sc = pltpu.async_copy(rows[b], agg_sh.at[dbuf[b]], ssem, add=True)
      if with_cnt:
        cc = pltpu.async_copy(ones_v, cnt_sh.at[dbuf[b]], csem, add=True)
      sc.wait()
      if with_cnt:
        cc.wait()
      if prefetch:
        prep(j + NBUF, b)

    with jax.named_scope("edge_loop"):
      for b in range(min(NBUF, cpt_pipe)):
        prep(b, b)

      def outer(jo, _):
        for b in range(NBUF):
          consume(jo * NBUF + b, b, True)
        return 0
      if cpt_pipe >= NBUF:
        lax.fori_loop(0, cpt_pipe // NBUF - 1, outer, 0)
        for b in range(NBUF):
          consume(cpt_pipe - NBUF + b, b, False)
      for j in range(cpt_pipe, cpt):
        prep(j, 0)
        consume(j, 0, False)
      if tail:
        toff = ebase + cpt * CHUNK
        pltpu.sync_copy(dst_hbm.at[pl.ds(toff, tail)], dtail)
        pltpu.async_copy(x_hbm.at[sidx.at[pl.ds(cpt * CHUNK, tail)]],
                         rows[0].at[pl.ds(0, tail)], gsem[0]).wait()
        pltpu.sync_copy(rows[0].at[pl.ds(0, tail)], agg_sh.at[dtail],
                        add=True)
        if with_cnt:
          pltpu.sync_copy(ones_v.at[pl.ds(0, tail)], cnt_sh.at[dtail],
                          add=True)
      plsc.subcore_barrier()

    with jax.named_scope("writeback"):
      def writeback(k, _):
        r = row0 + k * CHUNK
        pltpu.sync_copy(agg_sh.at[pl.ds(r, CHUNK)], rows[0])
        pltpu.sync_copy(rows[0], agg_out.at[c, pl.ds(r, CHUNK)])
        if with_cnt:
          pltpu.sync_copy(cnt_sh.at[pl.ds(r, CHUNK)], zrow_v)
          pltpu.sync_copy(zrow_v, cnt_out.at[c, pl.ds(r, CHUNK)])
        return 0
      lax.fori_loop(0, zchunks, writeback, 0)

  res = pl.kernel(
      body,
      out_type=tuple(out_types),
      mesh=mesh,
      scratch_types=scratch,
  )(x, src2d, dst2d)
  if not with_cnt and isinstance(res, (tuple, list)):
    res = res[0]
  return res


def _sage_linear(agg_parts, cnt_col, x, Wl, b, Wr, relu):
  n = x.shape[0]
  bn = 2000
  grid = n // bn

  def body(agg_ref, cnt_ref, x_ref, wl_ref, b_ref, wr_ref, o_ref):
    mean = (agg_ref[0] + agg_ref[1]) / jnp.maximum(cnt_ref[...], 1.0)
    o = (jnp.dot(mean, wl_ref[...], preferred_element_type=jnp.float32)
         + b_ref[...]
         + jnp.dot(x_ref[...], wr_ref[...], preferred_element_type=jnp.float32))
    if relu:
      o = jnp.maximum(o, 0.0)
    o_ref[...] = o

  return pl.pallas_call(
      body,
      grid=(grid,),
      in_specs=[
          pl.BlockSpec((NUM_CORES, bn, D), lambda i: (0, i, 0)),
          pl.BlockSpec((bn, 1), lambda i: (i, 0)),
          pl.BlockSpec((bn, D), lambda i: (i, 0)),
          pl.BlockSpec((D, D), lambda i: (0, 0)),
          pl.BlockSpec((1, D), lambda i: (0, 0)),
          pl.BlockSpec((D, D), lambda i: (0, 0)),
      ],
      out_specs=pl.BlockSpec((bn, D), lambda i: (i, 0)),
      out_shape=jax.ShapeDtypeStruct((n, D), jnp.float32),
  )(agg_parts, cnt_col, x, Wl, b.reshape(1, D), Wr)


def kernel(x, edge_index, W1l, b1, W1r, W2l, b2, W2r):
  n = x.shape[0]
  e = edge_index.shape[1]
  rows_align = NUM_SUBCORES * CHUNK
  npad = ((n + 1 + rows_align - 1) // rows_align) * rows_align
  e_align = NTILES * 8
  e_pad = ((e + e_align - 1) // e_align) * e_align

  src2d = edge_index[0].astype(jnp.int32)
  dst2d = edge_index[1].astype(jnp.int32)
  if e_pad != e:
    pad_i = jnp.arange(e_pad - e, dtype=jnp.int32)
    src2d = jnp.concatenate([src2d, pad_i % n])
    dst2d = jnp.concatenate([dst2d, pad_i % (npad - n) + n])

  agg1, cnt = _sage_aggregate(x, src2d, dst2d, npad, True)
  cnt_col = (cnt[0] + cnt[1]).reshape(npad, 1)
  h = _sage_linear(agg1, cnt_col, x, W1l, b1, W1r, relu=True)
  agg2 = _sage_aggregate(h, src2d, dst2d, npad, False)
  return _sage_linear(agg2, cnt_col, h, W2l, b2, W2r, relu=False)


# speedup vs baseline: 4.7198x; 1.0413x over previous
import jax
import jax.numpy as jnp
from jax import lax
from jax.experimental import pallas as pl
from jax.experimental.pallas import tpu as pltpu
from jax.experimental.pallas import tpu_sc as plsc

D = 128
CHUNK = 128
NBUF = 2
NUM_CORES = 2
NUM_SUBCORES = 16
NTILES = NUM_CORES * NUM_SUBCORES
LANES = 16


def _sage_aggregate(x, ei, npad, with_cnt):
  rows_per_tile = npad // NUM_SUBCORES
  zchunks = rows_per_tile // CHUNK
  e_all = ei.shape[0] // 2
  epc = e_all // NTILES
  cpt = epc // CHUNK
  tail = epc % CHUNK
  cpt_pipe = cpt - (cpt % NBUF)

  out_types = [jax.ShapeDtypeStruct((NUM_CORES, npad, D), jnp.float32)]
  if with_cnt:
    out_types.append(jax.ShapeDtypeStruct((NUM_CORES, npad), jnp.float32))

  scratch = (
      [pltpu.VMEM((CHUNK, D), jnp.float32) for _ in range(NBUF)]
      + [pltpu.VMEM((CHUNK,), jnp.int32) for _ in range(NBUF)]
      + [
          pltpu.VMEM((epc,), jnp.int32),
          pltpu.VMEM((max(tail, 1),), jnp.int32),
          pltpu.VMEM((CHUNK,), jnp.float32),
          pltpu.VMEM((CHUNK,), jnp.float32),
          pltpu.VMEM_SHARED((npad, D), jnp.float32),
          pltpu.VMEM_SHARED((npad,), jnp.float32),
      ]
      + [pltpu.SemaphoreType.DMA for _ in range(NBUF)]
      + [pltpu.SemaphoreType.DMA for _ in range(NBUF)]
      + [pltpu.SemaphoreType.DMA, pltpu.SemaphoreType.DMA]
  )
  mesh = plsc.VectorSubcoreMesh(core_axis_name="c", subcore_axis_name="s")

  def body(x_hbm, ei_hbm, *refs):
    it = iter(refs)
    agg_out = next(it)
    cnt_out = next(it) if with_cnt else None
    rows = [next(it) for _ in range(NBUF)]
    dbuf = [next(it) for _ in range(NBUF)]
    sidx = next(it)
    dtail = next(it)
    ones_v = next(it)
    zrow_v = next(it)
    agg_sh = next(it)
    cnt_sh = next(it)
    gsem = [next(it) for _ in range(NBUF)]
    dsem = [next(it) for _ in range(NBUF)]
    ssem = next(it)
    csem = next(it)

    c = lax.axis_index("c")
    s = lax.axis_index("s")
    wid = s * NUM_CORES + c
    row0 = s * rows_per_tile

    ebase = wid * epc
    dbase = e_all + ebase
    ssd = pltpu.async_copy(ei_hbm.at[pl.ds(ebase, epc)], sidx, gsem[0])

    def fill_rows(i, _):
      r = i // (D // LANES)
      col = (i % (D // LANES)) * LANES
      rows[0][r, pl.ds(col, LANES)] = jnp.zeros((LANES,), jnp.float32)
      return 0
    lax.fori_loop(0, CHUNK * (D // LANES), fill_rows, 0)

    def fill_small(i, _):
      ones_v[pl.ds(i * LANES, LANES)] = jnp.ones((LANES,), jnp.float32)
      zrow_v[pl.ds(i * LANES, LANES)] = jnp.zeros((LANES,), jnp.float32)
      return 0
    lax.fori_loop(0, CHUNK // LANES, fill_small, 0)

    with jax.named_scope("zero_acc"):
      def zero_blk(k, _):
        pltpu.sync_copy(rows[0], agg_sh.at[pl.ds(row0 + k * CHUNK, CHUNK)])
        pltpu.sync_copy(zrow_v, cnt_sh.at[pl.ds(row0 + k * CHUNK, CHUNK)])
        return 0
      lax.fori_loop(0, zchunks, zero_blk, 0)
      ssd.wait()
      plsc.subcore_barrier()

    def prep(j, b):
      pltpu.async_copy(ei_hbm.at[pl.ds(dbase + j * CHUNK, CHUNK)],
                       dbuf[b], dsem[b])
      pltpu.async_copy(x_hbm.at[sidx.at[pl.ds(j * CHUNK, CHUNK)]],
                       rows[b], gsem[b])

    def consume(j, b, prefetch):
      pltpu.make_async_copy(ei_hbm.at[pl.ds(dbase + j * CHUNK, CHUNK)],
                            dbuf[b], dsem[b]).wait()
      pltpu.make_async_copy(x_hbm.at[sidx.at[pl.ds(j * CHUNK, CHUNK)]],
                            rows[b], gsem[b]).wait()
      sc = pltpu.async_copy(rows[b], agg_sh.at[dbuf[b]], ssem, add=True)
      if with_cnt:
        cc = pltpu.async_copy(ones_v, cnt_sh.at[dbuf[b]], csem, add=True)
      sc.wait()
      if with_cnt:
        cc.wait()
      if prefetch:
        prep(j + NBUF, b)

    with jax.named_scope("edge_loop"):
      for b in range(min(NBUF, cpt_pipe)):
        prep(b, b)

      def outer(jo, _):
        for b in range(NBUF):
          consume(jo * NBUF + b, b, True)
        return 0
      if cpt_pipe >= NBUF:
        lax.fori_loop(0, cpt_pipe // NBUF - 1, outer, 0)
        for b in range(NBUF):
          consume(cpt_pipe - NBUF + b, b, False)
      for j in range(cpt_pipe, cpt):
        prep(j, 0)
        consume(j, 0, False)
      if tail:
        toff = cpt * CHUNK
        pltpu.sync_copy(ei_hbm.at[pl.ds(dbase + toff, tail)], dtail)
        pltpu.async_copy(x_hbm.at[sidx.at[pl.ds(toff, tail)]],
                         rows[0].at[pl.ds(0, tail)], gsem[0]).wait()
        pltpu.sync_copy(rows[0].at[pl.ds(0, tail)], agg_sh.at[dtail],
                        add=True)
        if with_cnt:
          pltpu.sync_copy(ones_v.at[pl.ds(0, tail)], cnt_sh.at[dtail],
                          add=True)
      plsc.subcore_barrier()

    with jax.named_scope("writeback"):
      def writeback(k, _):
        r = row0 + k * CHUNK
        pltpu.sync_copy(agg_sh.at[pl.ds(r, CHUNK)], rows[0])
        pltpu.sync_copy(rows[0], agg_out.at[c, pl.ds(r, CHUNK)])
        if with_cnt:
          pltpu.sync_copy(cnt_sh.at[pl.ds(r, CHUNK)], zrow_v)
          pltpu.sync_copy(zrow_v, cnt_out.at[c, pl.ds(r, CHUNK)])
        return 0
      lax.fori_loop(0, zchunks, writeback, 0)

  res = pl.kernel(
      body,
      out_type=tuple(out_types),
      mesh=mesh,
      scratch_types=scratch,
  )(x, ei)
  if not with_cnt and isinstance(res, (tuple, list)):
    res = res[0]
  return res


def _sage_linear(agg_parts, cnt_col, x, Wl, b, Wr, relu):
  n = x.shape[0]
  bn = 2000
  grid = n // bn

  def body(agg_ref, cnt_ref, x_ref, wl_ref, b_ref, wr_ref, o_ref):
    mean = (agg_ref[0] + agg_ref[1]) / jnp.maximum(cnt_ref[...], 1.0)
    o = (jnp.dot(mean, wl_ref[...], preferred_element_type=jnp.float32)
         + b_ref[...]
         + jnp.dot(x_ref[...], wr_ref[...], preferred_element_type=jnp.float32))
    if relu:
      o = jnp.maximum(o, 0.0)
    o_ref[...] = o

  return pl.pallas_call(
      body,
      grid=(grid,),
      in_specs=[
          pl.BlockSpec((NUM_CORES, bn, D), lambda i: (0, i, 0)),
          pl.BlockSpec((bn, 1), lambda i: (i, 0)),
          pl.BlockSpec((bn, D), lambda i: (i, 0)),
          pl.BlockSpec((D, D), lambda i: (0, 0)),
          pl.BlockSpec((1, D), lambda i: (0, 0)),
          pl.BlockSpec((D, D), lambda i: (0, 0)),
      ],
      out_specs=pl.BlockSpec((bn, D), lambda i: (i, 0)),
      out_shape=jax.ShapeDtypeStruct((n, D), jnp.float32),
  )(agg_parts, cnt_col, x, Wl, b.reshape(1, D), Wr)


def kernel(x, edge_index, W1l, b1, W1r, W2l, b2, W2r):
  n = x.shape[0]
  e = edge_index.shape[1]
  rows_align = NUM_SUBCORES * CHUNK
  npad = ((n + 1 + rows_align - 1) // rows_align) * rows_align
  e_align = NTILES * 8
  e_pad = ((e + e_align - 1) // e_align) * e_align

  ei = edge_index.astype(jnp.int32)
  if e_pad != e:
    pad_i = jnp.arange(e_pad - e, dtype=jnp.int32)
    ei = jnp.concatenate(
        [ei, jnp.stack([pad_i % n, pad_i % (npad - n) + n])], axis=1)
  ei = ei.reshape(2 * e_pad)

  agg1, cnt = _sage_aggregate(x, ei, npad, True)
  cnt_col = (cnt[0] + cnt[1]).reshape(npad, 1)
  h = _sage_linear(agg1, cnt_col, x, W1l, b1, W1r, relu=True)
  agg2 = _sage_aggregate(h, ei, npad, False)
  return _sage_linear(agg2, cnt_col, h, W2l, b2, W2r, relu=False)
